# baseline (speedup 1.0000x reference)
import functools

import jax
import jax.numpy as jnp
from jax import lax
from jax.experimental import pallas as pl
from jax.experimental.pallas import tpu as pltpu
from jax.experimental.pallas import tpu_sc as plsc

_TIME_INTERVAL = 10000.0
_N_TIME_INTERVAL = 100.0
_B = 4096
_L = 200
_EMB = 64
_NTOK = _B * _L


def _build(nw):
    rows_pw = _B // nw
    tok_pw = rows_pw * _L
    chunk = 512
    nchunk = tok_pw // chunk

    mesh = plsc.VectorSubcoreMesh(core_axis_name="c", subcore_axis_name="s")

    @functools.partial(
        pl.kernel,
        mesh=mesh,
        out_type=jax.ShapeDtypeStruct((_NTOK, _EMB), jnp.float32),
        scratch_types=[
            pltpu.VMEM((rows_pw, _L + 1), jnp.int32),
            pltpu.VMEM((tok_pw,), jnp.int32),
            pltpu.VMEM((chunk, _EMB), jnp.float32),
            pltpu.SemaphoreType.DMA,
        ],
        compiler_params=pltpu.CompilerParams(use_tc_tiling_on_sc=False),
    )
    def k(ts_hbm, table_hbm, out_hbm, ts_v, idx_v, rows_v, sem):
        wid = lax.axis_index("c") * 16 + lax.axis_index("s")
        row0 = wid * rows_pw
        tok0 = wid * tok_pw

        pltpu.sync_copy(ts_hbm.at[pl.ds(row0, rows_pw)], ts_v)

        def idx_body(r, carry):
            for i in range(13):
                c = 184 if i == 12 else i * 16
                t1 = ts_v[r, pl.ds(c + 1, 16)]
                t0 = ts_v[r, pl.ds(c, 16)]
                dt = (t1 - t0).astype(jnp.float32)
                bix = (dt / _TIME_INTERVAL * _N_TIME_INTERVAL).astype(jnp.int32)
                idx_v[pl.ds(r * _L + c, 16)] = jnp.maximum(bix, 0)
            return carry

        lax.fori_loop(0, rows_pw, idx_body, 0)

        def gather_body(g, carry):
            base = g * chunk
            cps = [
                pltpu.async_copy(
                    table_hbm.at[idx_v.at[pl.ds(base + j * 128, 128)]],
                    rows_v.at[pl.ds(j * 128, 128)],
                    sem,
                )
                for j in range(chunk // 128)
            ]
            for cp in cps:
                cp.wait()
            pltpu.sync_copy(rows_v, out_hbm.at[pl.ds(tok0 + base, chunk)])
            return carry

        lax.fori_loop(0, nchunk, gather_body, 0)

    return k


def kernel(inputs, timestamp, W, b):
    info = plsc.get_sparse_core_info()
    nw = info.num_cores * info.num_subcores
    table = (W.T + b[None, :]).astype(jnp.float32)
    out = _build(nw)(timestamp.astype(jnp.int32), table)
    return out.reshape(_B, _L, _EMB)


# speedup vs baseline: 19.6520x; 19.6520x over previous
import functools

import jax
import jax.numpy as jnp
from jax import lax
from jax.experimental import pallas as pl
from jax.experimental.pallas import tpu as pltpu
from jax.experimental.pallas import tpu_sc as plsc

_TIME_INTERVAL = 10000.0
_N_TIME_INTERVAL = 100.0
_B = 4096
_L = 200
_EMB = 64
_NTOK = _B * _L


def _build(nw):
    rows_pw = _B // nw
    tok_pw = rows_pw * _L
    chunk = 512
    nchunk = tok_pw // chunk

    mesh = plsc.VectorSubcoreMesh(core_axis_name="c", subcore_axis_name="s")

    @functools.partial(
        pl.kernel,
        mesh=mesh,
        out_type=jax.ShapeDtypeStruct((_NTOK, _EMB), jnp.float32),
        scratch_types=[
            pltpu.VMEM((rows_pw, _L + 1), jnp.int32),
            pltpu.VMEM((tok_pw,), jnp.int32),
            pltpu.VMEM((chunk, _EMB), jnp.float32),
            pltpu.VMEM_SHARED((101, _EMB), jnp.float32),
            pltpu.SemaphoreType.DMA,
        ],
        compiler_params=pltpu.CompilerParams(use_tc_tiling_on_sc=False),
    )
    def k(ts_hbm, table_hbm, out_hbm, ts_v, idx_v, rows_v, table_sh, sem):
        sid = lax.axis_index("s")
        wid = lax.axis_index("c") * 16 + sid
        row0 = wid * rows_pw
        tok0 = wid * tok_pw

        with jax.named_scope("stage_table"):
            @pl.when(sid == 0)
            def _():
                pltpu.sync_copy(table_hbm, table_sh)
            plsc.subcore_barrier()

        with jax.named_scope("stage_ts"):
            pltpu.sync_copy(ts_hbm.at[pl.ds(row0, rows_pw)], ts_v)

        def idx_body(r, carry):
            for i in range(13):
                c = 184 if i == 12 else i * 16
                t1 = ts_v[r, pl.ds(c + 1, 16)]
                t0 = ts_v[r, pl.ds(c, 16)]
                dt = (t1 - t0).astype(jnp.float32)
                bix = (dt / _TIME_INTERVAL * _N_TIME_INTERVAL).astype(jnp.int32)
                idx_v[pl.ds(r * _L + c, 16)] = jnp.maximum(bix, 0)
            return carry

        with jax.named_scope("compute_idx"):
            lax.fori_loop(0, rows_pw, idx_body, 0)

        def gather_body(g, carry):
            base = g * chunk
            with jax.named_scope("gather"):
                cps = [
                    pltpu.async_copy(
                        table_sh.at[idx_v.at[pl.ds(base + j * 128, 128)]],
                        rows_v.at[pl.ds(j * 128, 128)],
                        sem,
                    )
                    for j in range(chunk // 128)
                ]
                for cp in cps:
                    cp.wait()
            with jax.named_scope("writeout"):
                pltpu.sync_copy(rows_v, out_hbm.at[pl.ds(tok0 + base, chunk)])
            return carry

        with jax.named_scope("gather_loop"):
            lax.fori_loop(0, nchunk, gather_body, 0)

    return k


def kernel(inputs, timestamp, W, b):
    info = plsc.get_sparse_core_info()
    nw = info.num_cores * info.num_subcores
    table = (W.T + b[None, :]).astype(jnp.float32)
    out = _build(nw)(timestamp.astype(jnp.int32), table)
    return out.reshape(_B, _L, _EMB)


# speedup vs baseline: 19.9127x; 1.0133x over previous
import functools

import jax
import jax.numpy as jnp
from jax import lax
from jax.experimental import pallas as pl
from jax.experimental.pallas import tpu as pltpu
from jax.experimental.pallas import tpu_sc as plsc

_TIME_INTERVAL = 10000.0
_N_TIME_INTERVAL = 100.0
_B = 4096
_L = 200
_EMB = 64
_NTOK = _B * _L


def _build(nw):
    rows_pw = _B // nw
    tok_pw = rows_pw * _L
    chunk = 512
    nchunk = tok_pw // chunk
    npairs = nchunk // 2
    tsw = _L + 1

    mesh = plsc.VectorSubcoreMesh(core_axis_name="c", subcore_axis_name="s")

    @functools.partial(
        pl.kernel,
        mesh=mesh,
        out_type=jax.ShapeDtypeStruct((_NTOK, _EMB), jnp.float32),
        scratch_types=[
            pltpu.VMEM((rows_pw * tsw,), jnp.int32),
            pltpu.VMEM((tok_pw,), jnp.int32),
            pltpu.VMEM((chunk, _EMB), jnp.float32),
            pltpu.VMEM((chunk, _EMB), jnp.float32),
            pltpu.VMEM_SHARED((101, _EMB), jnp.float32),
            pltpu.SemaphoreType.DMA,
            pltpu.SemaphoreType.DMA,
            pltpu.SemaphoreType.DMA,
            pltpu.SemaphoreType.DMA,
        ],
        compiler_params=pltpu.CompilerParams(use_tc_tiling_on_sc=False),
    )
    def k(ts_hbm, table_hbm, out_hbm, ts_v, idx_v, rows0, rows1,
          table_sh, gsem0, gsem1, osem0, osem1):
        sid = lax.axis_index("s")
        wid = lax.axis_index("c") * 16 + sid
        tok0 = wid * tok_pw

        @pl.when(sid == 0)
        def _():
            pltpu.sync_copy(table_hbm, table_sh)

        pltpu.sync_copy(ts_hbm.at[pl.ds(wid * rows_pw * tsw, rows_pw * tsw)],
                        ts_v)

        def idx_body(r, carry):
            for i in range(13):
                c = 184 if i == 12 else i * 16
                t1 = ts_v[pl.ds(r * tsw + c + 1, 16)]
                t0 = ts_v[pl.ds(r * tsw + c, 16)]
                dt = (t1 - t0).astype(jnp.float32)
                bix = (dt / _TIME_INTERVAL * _N_TIME_INTERVAL).astype(jnp.int32)
                idx_v[pl.ds(r * _L + c, 16)] = jnp.maximum(bix, 0)
            return carry

        lax.fori_loop(0, rows_pw, idx_body, 0)

        plsc.subcore_barrier()

        def gathers(g, rows, sem):
            for j in range(chunk // 128):
                pltpu.async_copy(
                    table_sh.at[idx_v.at[pl.ds(g * chunk + j * 128, 128)]],
                    rows.at[pl.ds(j * 128, 128)],
                    sem,
                )

        def drain_gather(rows, sem):
            pltpu.make_async_copy(out_hbm.at[pl.ds(0, chunk)], rows, sem).wait()

        def drain_write(sem):
            pltpu.make_async_copy(rows0, out_hbm.at[pl.ds(0, chunk)], sem).wait()

        gathers(0, rows0, gsem0)

        def pair_body(gg, carry):
            g0 = gg * 2
            g1 = g0 + 1
            drain_gather(rows0, gsem0)

            @pl.when(gg > 0)
            def _():
                drain_write(osem1)
            gathers(g1, rows1, gsem1)
            pltpu.async_copy(rows0, out_hbm.at[pl.ds(tok0 + g0 * chunk, chunk)],
                             osem0)

            drain_gather(rows1, gsem1)
            drain_write(osem0)

            @pl.when(gg + 1 < npairs)
            def _():
                gathers(g0 + 2, rows0, gsem0)
            pltpu.async_copy(rows1, out_hbm.at[pl.ds(tok0 + g1 * chunk, chunk)],
                             osem1)
            return carry

        lax.fori_loop(0, npairs, pair_body, 0)
        drain_write(osem1)

    return k


def kernel(inputs, timestamp, W, b):
    info = plsc.get_sparse_core_info()
    nw = info.num_cores * info.num_subcores
    table = (W.T + b[None, :]).astype(jnp.float32)
    ts_flat = timestamp.astype(jnp.int32).reshape(-1)
    out = _build(nw)(ts_flat, table)
    return out.reshape(_B, _L, _EMB)
